# Initial kernel scaffold; baseline (speedup 1.0000x reference)
import jax
import jax.numpy as jnp
from jax.experimental import pallas as pl


def kernel(cate_x, mask, table, gamma, beta):
    raise NotImplementedError("write your pallas kernel here")


# speedup vs baseline: 1.8762x; 1.8762x over previous
import functools

import jax
import jax.numpy as jnp
from jax import lax
from jax.experimental import pallas as pl
from jax.experimental.pallas import tpu as pltpu
from jax.experimental.pallas import tpu_sc as plsc

B = 1024
T = 50
NF = 26
FIELD_V = 38461
EMB = 16
NORM_DIM = NF * EMB
EPS = 1e-5

NTOK = B * T
NW = 32
TOK_W = NTOK // NW
C = 64
NCHUNK = TOK_W // C
ROWS_PER_CHUNK = C * NF
NDMA = ROWS_PER_CHUNK // 128
GROUPS = C // 16


def _sc_body(idx_hbm, table_hbm, gamma_hbm, beta_hbm, out_hbm,
             idx_v, rows_v, gamma_v, beta_v, sem):
    wid = lax.axis_index("s") * 2 + lax.axis_index("c")
    pltpu.sync_copy(gamma_hbm, gamma_v)
    pltpu.sync_copy(beta_hbm, beta_v)
    idx_base0 = wid * TOK_W * NF
    out_row0 = wid * TOK_W * NF

    lane = jnp.arange(16, dtype=jnp.int32)

    def chunk_body(k, carry):
        pltpu.sync_copy(
            idx_hbm.at[pl.ds(idx_base0 + k * ROWS_PER_CHUNK, ROWS_PER_CHUNK)],
            idx_v)
        copies = [
            pltpu.make_async_copy(
                table_hbm.at[idx_v.at[pl.ds(j * 128, 128)]],
                rows_v.at[pl.ds(j * 128, 128)],
                sem,
            )
            for j in range(NDMA)
        ]
        for cp in copies:
            cp.start()
        for cp in copies:
            cp.wait()

        def group_body(g, carry2):
            base_row = (lane + g * 16) * NF

            def pass1(f, acc):
                s1, s2 = acc
                r = base_row + f
                for u in range(16):
                    cu = jnp.full((16,), u, dtype=jnp.int32)
                    v = plsc.load_gather(rows_v, [r, cu])
                    s1 = s1 + v
                    s2 = s2 + v * v
                return (s1, s2)

            zero = jnp.zeros((16,), jnp.float32)
            s1, s2 = lax.fori_loop(0, NF, pass1, (zero, zero))
            mean = s1 * (1.0 / NORM_DIM)
            var = s2 * (1.0 / NORM_DIM) - mean * mean
            x = var + EPS
            i = jnp.int32(0x5F3759DF) - lax.shift_right_arithmetic(
                plsc.bitcast(x, jnp.int32), 1)
            y = plsc.bitcast(i, jnp.float32)
            for _ in range(3):
                y = y * (1.5 - 0.5 * x * y * y)
            rstd = y

            def pass2(f, _):
                r = base_row + f
                gvec = gamma_v[pl.ds(f * 16, 16)]
                bvec = beta_v[pl.ds(f * 16, 16)]
                for u in range(16):
                    cu = jnp.full((16,), u, dtype=jnp.int32)
                    v = plsc.load_gather(rows_v, [r, cu])
                    gb = gvec.at[cu].get(mode="promise_in_bounds")
                    bb = bvec.at[cu].get(mode="promise_in_bounds")
                    o = (v - mean) * rstd
                    o = o * gb + bb
                    plsc.store_scatter(rows_v, [r, cu], o)
                return 0

            lax.fori_loop(0, NF, pass2, 0)
            return carry2

        lax.fori_loop(0, GROUPS, group_body, 0)
        pltpu.sync_copy(
            rows_v,
            out_hbm.at[pl.ds(out_row0 + k * ROWS_PER_CHUNK, ROWS_PER_CHUNK)])
        return carry

    lax.fori_loop(0, NCHUNK, chunk_body, 0)


@jax.jit
def _sc_call(shifted2d, table, gamma, beta):
    mesh = plsc.VectorSubcoreMesh(core_axis_name="c", subcore_axis_name="s")
    f = pl.kernel(
        _sc_body,
        out_type=jax.ShapeDtypeStruct((NTOK * NF, EMB), jnp.float32),
        mesh=mesh,
        scratch_types=[
            pltpu.VMEM((ROWS_PER_CHUNK,), jnp.int32),
            pltpu.VMEM((ROWS_PER_CHUNK, EMB), jnp.float32),
            pltpu.VMEM((NORM_DIM,), jnp.float32),
            pltpu.VMEM((NORM_DIM,), jnp.float32),
            pltpu.SemaphoreType.DMA,
        ],
        compiler_params=pltpu.CompilerParams(
            needs_layout_passes=False, use_tc_tiling_on_sc=False),
    )
    return f(shifted2d, table, gamma, beta)


def kernel(cate_x, mask, table, gamma, beta):
    offsets = jnp.arange(NF, dtype=cate_x.dtype) * FIELD_V
    shifted = cate_x + mask[:, :, None] * offsets[None, None, :]
    shifted1d = shifted.reshape(NTOK * NF)
    out = _sc_call(shifted1d, table, gamma, beta)
    return out.reshape(B, T, NORM_DIM)


# speedup vs baseline: 3.2152x; 1.7137x over previous
import functools

import jax
import jax.numpy as jnp
from jax import lax
from jax.experimental import pallas as pl
from jax.experimental.pallas import tpu as pltpu
from jax.experimental.pallas import tpu_sc as plsc

B = 1024
T = 50
NF = 26
FIELD_V = 38461
EMB = 16
NORM_DIM = NF * EMB
EPS = 1e-5

NTOK = B * T
NW = 32
TOK_W = NTOK // NW
C = 64
NCHUNK = TOK_W // C
ROWS_PER_CHUNK = C * NF
NDMA = ROWS_PER_CHUNK // 128
GROUPS = C // 16


def _sc_body(idx_hbm, table_hbm, gamma_hbm, beta_hbm, out_hbm,
             idx_v, rows_v, gamma_v, beta_v, sem):
    wid = lax.axis_index("s") * 2 + lax.axis_index("c")
    pltpu.sync_copy(gamma_hbm, gamma_v)
    pltpu.sync_copy(beta_hbm, beta_v)
    idx_base0 = wid * TOK_W * NF
    out_row0 = wid * TOK_W * NF

    lane = jnp.arange(16, dtype=jnp.int32)

    def chunk_body(k, carry):
        pltpu.sync_copy(
            idx_hbm.at[pl.ds(idx_base0 + k * ROWS_PER_CHUNK, ROWS_PER_CHUNK)],
            idx_v)
        copies = [
            pltpu.make_async_copy(
                table_hbm.at[idx_v.at[pl.ds(j * 128, 128)]],
                rows_v.at[pl.ds(j * 128, 128)],
                sem,
            )
            for j in range(NDMA)
        ]
        for cp in copies:
            cp.start()
        for cp in copies:
            cp.wait()

        skew = [jnp.bitwise_and(lane + u, 15) for u in range(16)]

        def group_body(g, carry2):
            base_row = (lane + g * 16) * NF

            def pass1(f, acc):
                s1a, s1b, s1c, s1d, s2a, s2b, s2c, s2d = acc
                r = base_row + f
                s1 = [s1a, s1b, s1c, s1d]
                s2 = [s2a, s2b, s2c, s2d]
                for u in range(16):
                    v = plsc.load_gather(rows_v, [r, skew[u]])
                    s1[u % 4] = s1[u % 4] + v
                    s2[u % 4] = s2[u % 4] + v * v
                return (*s1, *s2)

            zero = jnp.zeros((16,), jnp.float32)
            accs = lax.fori_loop(0, NF, pass1, (zero,) * 8)
            s1 = (accs[0] + accs[1]) + (accs[2] + accs[3])
            s2 = (accs[4] + accs[5]) + (accs[6] + accs[7])
            mean = s1 * (1.0 / NORM_DIM)
            var = s2 * (1.0 / NORM_DIM) - mean * mean
            x = var + EPS
            i = jnp.int32(0x5F3759DF) - lax.shift_right_arithmetic(
                plsc.bitcast(x, jnp.int32), 1)
            y = plsc.bitcast(i, jnp.float32)
            for _ in range(3):
                y = y * (1.5 - 0.5 * x * y * y)
            rstd = y

            def pass2(f, _):
                r = base_row + f
                gvec = gamma_v[pl.ds(f * 16, 16)]
                bvec = beta_v[pl.ds(f * 16, 16)]
                for u in range(16):
                    v = plsc.load_gather(rows_v, [r, skew[u]])
                    gb = gvec.at[skew[u]].get(mode="promise_in_bounds")
                    bb = bvec.at[skew[u]].get(mode="promise_in_bounds")
                    o = (v - mean) * rstd
                    o = o * gb + bb
                    plsc.store_scatter(rows_v, [r, skew[u]], o)
                return 0

            lax.fori_loop(0, NF, pass2, 0)
            return carry2

        lax.fori_loop(0, GROUPS, group_body, 0)
        pltpu.sync_copy(
            rows_v,
            out_hbm.at[pl.ds(out_row0 + k * ROWS_PER_CHUNK, ROWS_PER_CHUNK)])
        return carry

    lax.fori_loop(0, NCHUNK, chunk_body, 0)


@jax.jit
def _sc_call(shifted2d, table, gamma, beta):
    mesh = plsc.VectorSubcoreMesh(core_axis_name="c", subcore_axis_name="s")
    f = pl.kernel(
        _sc_body,
        out_type=jax.ShapeDtypeStruct((NTOK * NF, EMB), jnp.float32),
        mesh=mesh,
        scratch_types=[
            pltpu.VMEM((ROWS_PER_CHUNK,), jnp.int32),
            pltpu.VMEM((ROWS_PER_CHUNK, EMB), jnp.float32),
            pltpu.VMEM((NORM_DIM,), jnp.float32),
            pltpu.VMEM((NORM_DIM,), jnp.float32),
            pltpu.SemaphoreType.DMA,
        ],
        compiler_params=pltpu.CompilerParams(
            needs_layout_passes=False, use_tc_tiling_on_sc=False),
    )
    return f(shifted2d, table, gamma, beta)


def kernel(cate_x, mask, table, gamma, beta):
    offsets = jnp.arange(NF, dtype=cate_x.dtype) * FIELD_V
    shifted = cate_x + mask[:, :, None] * offsets[None, None, :]
    shifted1d = shifted.reshape(NTOK * NF)
    out = _sc_call(shifted1d, table, gamma, beta)
    return out.reshape(B, T, NORM_DIM)
